# baseline (speedup 1.0000x reference)
import functools

import jax
import jax.numpy as jnp
from jax.experimental import pallas as pl
from jax.experimental.pallas import tpu as pltpu

N = 2048
K = 64
BM = 128
OBS_RADIUS_SQ = 1.0


def _mlp_body(states_ref, goals_ref, sel_ref, w1_ref, b1_ref, w2_ref, b2_ref,
              wd1a_ref, wd1b_ref, bd1_ref, wd2_ref, bd2_ref, wd3_ref, bd3_ref,
              wd4_ref, bd4_ref, out_ref):
    s = states_ref[...]
    g = goals_ref[...]
    sel = sel_ref[...]
    w1 = w1_ref[...]
    b1 = b1_ref[...]

    x0 = s[:, 0:1] - sel[:, 0, :]
    x1 = s[:, 1:2] - sel[:, 1, :]
    x2 = s[:, 2:3] - sel[:, 2, :]
    x3 = s[:, 3:4] - sel[:, 3, :]
    ind = sel[:, 4, :]
    d2 = x0 * x0 + x1 * x1
    mask = (d2 < OBS_RADIUS_SQ).astype(jnp.float32)

    h1 = (b1.reshape(1, 1, 64)
          + x0[:, :, None] * w1[0, :][None, None, :]
          + x1[:, :, None] * w1[1, :][None, None, :]
          + x2[:, :, None] * w1[2, :][None, None, :]
          + x3[:, :, None] * w1[3, :][None, None, :]
          + ind[:, :, None] * w1[4, :][None, None, :])
    h1 = jnp.maximum(h1, 0.0)
    h1f = h1.reshape(BM * K, 64)

    h2 = jnp.dot(h1f, w2_ref[...], preferred_element_type=jnp.float32)
    h2 = jnp.maximum(h2 + b2_ref[...], 0.0)
    h2 = h2.reshape(BM, K, 128) * mask[:, :, None]
    xl = jnp.max(h2, axis=1)

    e4 = jnp.concatenate([s[:, :2] - g, s[:, 2:]], axis=1)
    d1 = jnp.dot(xl, wd1a_ref[...], preferred_element_type=jnp.float32)
    d1 = d1 + jnp.dot(e4, wd1b_ref[...], preferred_element_type=jnp.float32)
    d1 = jnp.maximum(d1 + bd1_ref[...], 0.0)
    dd = jnp.dot(d1, wd2_ref[...], preferred_element_type=jnp.float32)
    dd = jnp.maximum(dd + bd2_ref[...], 0.0)
    dd = jnp.dot(dd, wd3_ref[...], preferred_element_type=jnp.float32)
    dd = jnp.maximum(dd + bd3_ref[...], 0.0)
    xo = jnp.dot(dd, wd4_ref[...], preferred_element_type=jnp.float32)
    xo = xo + bd4_ref[...]
    sg = 2.0 * jax.nn.sigmoid(xo) + 0.2
    ax = -(sg[:, 0] * e4[:, 0] + sg[:, 1] * e4[:, 2])
    ay = -(sg[:, 2] * e4[:, 1] + sg[:, 3] * e4[:, 3])
    out_ref[...] = jnp.stack([ax, ay], axis=1)


def _full(shape):
    return pl.BlockSpec(shape, lambda i: (0,) * len(shape))


@functools.partial(jax.jit, static_argnames=("interpret",))
def _mlp_call(states, goals, sel5, Wc1, bc1, Wc2, bc2, Wd1, bd1, Wd2, bd2,
              Wd3, bd3, Wd4, bd4, interpret=False):
    grid = (N // BM,)
    return pl.pallas_call(
        _mlp_body,
        grid=grid,
        in_specs=[
            pl.BlockSpec((BM, 4), lambda i: (i, 0)),
            pl.BlockSpec((BM, 2), lambda i: (i, 0)),
            pl.BlockSpec((BM, 5, K), lambda i: (i, 0, 0)),
            _full((5, 64)), _full((1, 64)),
            _full((64, 128)), _full((1, 128)),
            _full((128, 64)), _full((4, 64)), _full((1, 64)),
            _full((64, 128)), _full((1, 128)),
            _full((128, 64)), _full((1, 64)),
            _full((64, 4)), _full((1, 4)),
        ],
        out_specs=pl.BlockSpec((BM, 2), lambda i: (i, 0)),
        out_shape=jax.ShapeDtypeStruct((N, 2), jnp.float32),
        interpret=interpret,
    )(states, goals, sel5, Wc1, bc1.reshape(1, 64), Wc2, bc2.reshape(1, 128),
      Wd1[:128], Wd1[128:], bd1.reshape(1, 64), Wd2, bd2.reshape(1, 128),
      Wd3, bd3.reshape(1, 64), Wd4, bd4.reshape(1, 4))


def _select_gather_xla(states):
    p = states[:, :2]
    diff = p[:, None, :] - p[None, :, :]
    dist = jnp.linalg.norm(diff, axis=2)
    _, idx = jax.lax.top_k(-dist, K)
    sel_states = jnp.take(states, idx, axis=0)
    ind = (idx == jnp.arange(N)[:, None]).astype(jnp.float32)
    sel5 = jnp.concatenate(
        [jnp.transpose(sel_states, (0, 2, 1)), ind[:, None, :]], axis=1)
    return sel5


def kernel(states, goals, Wc1, bc1, Wc2, bc2, Wd1, bd1, Wd2, bd2, Wd3, bd3,
           Wd4, bd4):
    sel5 = _select_gather_xla(states)
    return _mlp_call(states, goals, sel5, Wc1, bc1, Wc2, bc2, Wd1, bd1,
                     Wd2, bd2, Wd3, bd3, Wd4, bd4)


# speedup vs baseline: 3.0151x; 3.0151x over previous
import functools

import jax
import jax.numpy as jnp
from jax import lax
from jax.experimental import pallas as pl
from jax.experimental.pallas import tpu as pltpu
from jax.experimental.pallas import tpu_sc as plsc

N = 2048
K = 64
BM = 128
OBS_RADIUS_SQ = 1.0

NW = 32
ROWS_PER_W = N // NW
NCHUNK = N // 16
KCH = K // 16


def _sc_body(statesT_hbm, out_hbm, ch_ref, cand_d_ref, cand_i_ref,
             surv_a_ref, surv_b_ref, hist_ref, sel_ref, row_ref):
    cid = lax.axis_index("c")
    sid = lax.axis_index("s")
    wid = sid * 2 + cid
    pltpu.sync_copy(statesT_hbm, ch_ref)
    lane = lax.iota(jnp.int32, 16)
    ones = jnp.ones((16,), jnp.int32)

    def row_body(rr, _carry):
        r = wid * ROWS_PER_W + rr
        rsplat = jnp.full((16,), r, jnp.int32)
        xi = plsc.load_gather(ch_ref, [rsplat])
        yi = plsc.load_gather(ch_ref, [rsplat + N])

        def p1(j, cur):
            base = j * 16
            dx = ch_ref[pl.ds(base, 16)] - xi
            dy = ch_ref[pl.ds(N + base, 16)] - yi
            d2 = dx * dx + dy * dy
            m = d2 < OBS_RADIUS_SQ
            mi = m.astype(jnp.int32)
            pos = cur + plsc.cumsum(mi) - 1
            plsc.store_scatter(cand_i_ref, [pos], lane + base, mask=m)
            plsc.store_scatter(cand_d_ref, [pos], d2, mask=m)
            return cur + jnp.sum(mi)

        c = lax.fori_loop(0, NCHUNK, p1, 0, unroll=2)
        kk = jnp.minimum(K, c)

        def level(in_ref, out_ref, state, shift):
            cur_n, want, tbits, nlt_tot = state
            cc = (cur_n + 15) // 16
            for l in range(16):
                hist_ref[pl.ds(l * 16, 16)] = jnp.zeros((16,), jnp.int32)

            def hb(j, carry):
                v = in_ref[pl.ds(j * 16, 16)]
                b = plsc.bitcast(v, jnp.int32)
                digit = (b >> shift) & 15
                lm = (lane + j * 16) < cur_n
                plsc.addupdate_scatter(hist_ref, [lane * 16 + digit], ones,
                                       mask=lm)
                return carry

            lax.fori_loop(0, cc, hb, 0)
            tot = hist_ref[pl.ds(0, 16)]
            for l in range(1, 16):
                tot = tot + hist_ref[pl.ds(l * 16, 16)]
            cum = plsc.cumsum(tot)
            below = cum < want
            bsel = jnp.sum(below.astype(jnp.int32))
            nlt = jnp.sum(jnp.where(below, tot, 0))

            def cb(j, cur2):
                v = in_ref[pl.ds(j * 16, 16)]
                b = plsc.bitcast(v, jnp.int32)
                digit = (b >> shift) & 15
                lm = (lane + j * 16) < cur_n
                m = (digit == bsel) & lm
                mi = m.astype(jnp.int32)
                pos = cur2 + plsc.cumsum(mi) - 1
                plsc.store_scatter(out_ref, [pos], v, mask=m)
                return cur2 + jnp.sum(mi)

            nn = lax.fori_loop(0, cc, cb, 0)
            return (nn, want - nlt, tbits | (bsel << shift), nlt_tot + nlt)

        st = (c, kk, 0, 0)
        bufs = (cand_d_ref, surv_a_ref, surv_b_ref)
        inb = 0
        for lev in range(8):
            outb = 1 if inb != 1 else 2
            st = level(bufs[inb], bufs[outb], st, 28 - 4 * lev)
            inb = outb
        _, _, tbits, nlt_tot = st

        for q in range(KCH):
            sel_ref[pl.ds(q * 16, 16)] = jnp.broadcast_to(r, (16,)).astype(
                jnp.int32)

        def eb(j, carry):
            clt, ceq = carry
            v = cand_d_ref[pl.ds(j * 16, 16)]
            b = plsc.bitcast(v, jnp.int32)
            iv = cand_i_ref[pl.ds(j * 16, 16)]
            lm = (lane + j * 16) < c
            mlt = (b < tbits) & lm
            mlti = mlt.astype(jnp.int32)
            plsc.store_scatter(sel_ref, [clt + plsc.cumsum(mlti) - 1], iv,
                               mask=mlt)
            meq = (b == tbits) & lm
            meqi = meq.astype(jnp.int32)
            pe = nlt_tot + ceq + plsc.cumsum(meqi) - 1
            keep = meq & (pe < kk)
            plsc.store_scatter(sel_ref, [pe], iv, mask=keep)
            return (clt + jnp.sum(mlti), ceq + jnp.sum(meqi))

        lax.fori_loop(0, (c + 15) // 16, eb, (0, 0))

        for q in range(KCH):
            iv = sel_ref[pl.ds(q * 16, 16)]
            for cch in range(4):
                row_ref[pl.ds(cch * K + q * 16, 16)] = plsc.load_gather(
                    ch_ref, [iv + cch * N])
            row_ref[pl.ds(4 * K + q * 16, 16)] = jnp.where(
                iv == r, 1.0, 0.0).astype(jnp.float32)
        pltpu.sync_copy(row_ref, out_hbm.at[r])
        return _carry

    lax.fori_loop(0, ROWS_PER_W, row_body, 0)


def _sc_select(statesT_flat):
    mesh = plsc.VectorSubcoreMesh(core_axis_name="c", subcore_axis_name="s")
    f = pl.kernel(
        _sc_body,
        out_type=jax.ShapeDtypeStruct((N, 5 * K), jnp.float32),
        mesh=mesh,
        compiler_params=pltpu.CompilerParams(needs_layout_passes=False),
        scratch_types=[
            pltpu.VMEM((4 * N,), jnp.float32),
            pltpu.VMEM((N,), jnp.float32),
            pltpu.VMEM((N,), jnp.int32),
            pltpu.VMEM((N,), jnp.float32),
            pltpu.VMEM((N,), jnp.float32),
            pltpu.VMEM((256,), jnp.int32),
            pltpu.VMEM((K,), jnp.int32),
            pltpu.VMEM((5 * K,), jnp.float32),
        ],
    )
    return f(statesT_flat)


def _mlp_body(states_ref, goals_ref, sel_ref, w1_ref, b1_ref, w2_ref, b2_ref,
              wd1a_ref, wd1b_ref, bd1_ref, wd2_ref, bd2_ref, wd3_ref, bd3_ref,
              wd4_ref, bd4_ref, out_ref):
    s = states_ref[...]
    g = goals_ref[...]
    sel = sel_ref[...]
    w1 = w1_ref[...]
    b1 = b1_ref[...]

    x0 = s[:, 0:1] - sel[:, 0, :]
    x1 = s[:, 1:2] - sel[:, 1, :]
    x2 = s[:, 2:3] - sel[:, 2, :]
    x3 = s[:, 3:4] - sel[:, 3, :]
    ind = sel[:, 4, :]
    d2 = x0 * x0 + x1 * x1
    mask = (d2 < OBS_RADIUS_SQ).astype(jnp.float32)

    h1 = (b1.reshape(1, 1, 64)
          + x0[:, :, None] * w1[0, :][None, None, :]
          + x1[:, :, None] * w1[1, :][None, None, :]
          + x2[:, :, None] * w1[2, :][None, None, :]
          + x3[:, :, None] * w1[3, :][None, None, :]
          + ind[:, :, None] * w1[4, :][None, None, :])
    h1 = jnp.maximum(h1, 0.0)
    h1f = h1.reshape(BM * K, 64)

    h2 = jnp.dot(h1f, w2_ref[...], preferred_element_type=jnp.float32)
    h2 = jnp.maximum(h2 + b2_ref[...], 0.0)
    h2 = h2.reshape(BM, K, 128) * mask[:, :, None]
    xl = jnp.max(h2, axis=1)

    e4 = jnp.concatenate([s[:, :2] - g, s[:, 2:]], axis=1)
    d1 = jnp.dot(xl, wd1a_ref[...], preferred_element_type=jnp.float32)
    d1 = d1 + jnp.dot(e4, wd1b_ref[...], preferred_element_type=jnp.float32)
    d1 = jnp.maximum(d1 + bd1_ref[...], 0.0)
    dd = jnp.dot(d1, wd2_ref[...], preferred_element_type=jnp.float32)
    dd = jnp.maximum(dd + bd2_ref[...], 0.0)
    dd = jnp.dot(dd, wd3_ref[...], preferred_element_type=jnp.float32)
    dd = jnp.maximum(dd + bd3_ref[...], 0.0)
    xo = jnp.dot(dd, wd4_ref[...], preferred_element_type=jnp.float32)
    xo = xo + bd4_ref[...]
    sg = 2.0 * jax.nn.sigmoid(xo) + 0.2
    ax = -(sg[:, 0] * e4[:, 0] + sg[:, 1] * e4[:, 2])
    ay = -(sg[:, 2] * e4[:, 1] + sg[:, 3] * e4[:, 3])
    out_ref[...] = jnp.stack([ax, ay], axis=1)


def _full(shape):
    return pl.BlockSpec(shape, lambda i: (0,) * len(shape))


@functools.partial(jax.jit, static_argnames=("interpret",))
def _mlp_call(states, goals, sel5, Wc1, bc1, Wc2, bc2, Wd1, bd1, Wd2, bd2,
              Wd3, bd3, Wd4, bd4, interpret=False):
    grid = (N // BM,)
    return pl.pallas_call(
        _mlp_body,
        grid=grid,
        in_specs=[
            pl.BlockSpec((BM, 4), lambda i: (i, 0)),
            pl.BlockSpec((BM, 2), lambda i: (i, 0)),
            pl.BlockSpec((BM, 5, K), lambda i: (i, 0, 0)),
            _full((5, 64)), _full((1, 64)),
            _full((64, 128)), _full((1, 128)),
            _full((128, 64)), _full((4, 64)), _full((1, 64)),
            _full((64, 128)), _full((1, 128)),
            _full((128, 64)), _full((1, 64)),
            _full((64, 4)), _full((1, 4)),
        ],
        out_specs=pl.BlockSpec((BM, 2), lambda i: (i, 0)),
        out_shape=jax.ShapeDtypeStruct((N, 2), jnp.float32),
        interpret=interpret,
    )(states, goals, sel5, Wc1, bc1.reshape(1, 64), Wc2, bc2.reshape(1, 128),
      Wd1[:128], Wd1[128:], bd1.reshape(1, 64), Wd2, bd2.reshape(1, 128),
      Wd3, bd3.reshape(1, 64), Wd4, bd4.reshape(1, 4))


def kernel(states, goals, Wc1, bc1, Wc2, bc2, Wd1, bd1, Wd2, bd2, Wd3, bd3,
           Wd4, bd4):
    sel5 = _sc_select(states.T.reshape(4 * N)).reshape(N, 5, K)
    return _mlp_call(states, goals, sel5, Wc1, bc1, Wc2, bc2, Wd1, bd1,
                     Wd2, bd2, Wd3, bd3, Wd4, bd4)


# speedup vs baseline: 3.3550x; 1.1127x over previous
import functools

import jax
import jax.numpy as jnp
from jax import lax
from jax.experimental import pallas as pl
from jax.experimental.pallas import tpu as pltpu
from jax.experimental.pallas import tpu_sc as plsc

N = 2048
K = 64
BM = 128
OBS_RADIUS_SQ = 1.0

NW = 32
ROWS_PER_W = N // NW
NCHUNK = N // 16
KCH = K // 16


def _sc_body(statesT_hbm, out_hbm, ch_ref, cand_d_ref, cand_i_ref,
             surv_a_ref, surv_b_ref, hist_ref, sel_ref, row_ref):
    cid = lax.axis_index("c")
    sid = lax.axis_index("s")
    wid = sid * 2 + cid
    pltpu.sync_copy(statesT_hbm, ch_ref)
    lane = lax.iota(jnp.int32, 16)
    ones = jnp.ones((16,), jnp.int32)

    def row_body(rr, _carry):
        r = wid * ROWS_PER_W + rr
        rsplat = jnp.full((16,), r, jnp.int32)
        xi = plsc.load_gather(ch_ref, [rsplat])
        yi = plsc.load_gather(ch_ref, [rsplat + N])

        def p1(j, cur):
            base = j * 16
            dx = ch_ref[pl.ds(base, 16)] - xi
            dy = ch_ref[pl.ds(N + base, 16)] - yi
            d2 = dx * dx + dy * dy
            m = d2 < OBS_RADIUS_SQ
            plsc.store_compressed(cand_i_ref.at[pl.ds(cur, 16)], lane + base,
                                  mask=m)
            plsc.store_compressed(cand_d_ref.at[pl.ds(cur, 16)], d2, mask=m)
            return cur + plsc.all_reduce_population_count(m)[0]

        c = lax.fori_loop(0, NCHUNK, p1, 0, unroll=2)
        kk = jnp.minimum(K, c)

        def level(in_ref, out_ref, state, shift):
            cur_n, want, tbits, nlt_tot = state
            cc = (cur_n + 15) // 16
            for l in range(16):
                hist_ref[pl.ds(l * 16, 16)] = jnp.zeros((16,), jnp.int32)

            def hb(j, carry):
                v = in_ref[pl.ds(j * 16, 16)]
                b = plsc.bitcast(v, jnp.int32)
                digit = (b >> shift) & 15
                lm = (lane + j * 16) < cur_n
                plsc.addupdate_scatter(hist_ref, [lane * 16 + digit], ones,
                                       mask=lm)
                return carry

            lax.fori_loop(0, cc, hb, 0)
            tot = hist_ref[pl.ds(0, 16)]
            for l in range(1, 16):
                tot = tot + hist_ref[pl.ds(l * 16, 16)]
            cum = plsc.cumsum(tot)
            below = cum < want
            bsel = plsc.all_reduce_population_count(below)[0]
            nlt = jnp.sum(jnp.where(below, tot, 0))

            def cb(j, cur2):
                v = in_ref[pl.ds(j * 16, 16)]
                b = plsc.bitcast(v, jnp.int32)
                digit = (b >> shift) & 15
                lm = (lane + j * 16) < cur_n
                m = (digit == bsel) & lm
                plsc.store_compressed(out_ref.at[pl.ds(cur2, 16)], v, mask=m)
                return cur2 + plsc.all_reduce_population_count(m)[0]

            nn = lax.fori_loop(0, cc, cb, 0)
            return (nn, want - nlt, tbits | (bsel << shift), nlt_tot + nlt)

        st = (c, kk, 0, 0)
        bufs = (cand_d_ref, surv_a_ref, surv_b_ref)
        inb = 0
        for lev in range(8):
            outb = 1 if inb != 1 else 2
            st = level(bufs[inb], bufs[outb], st, 28 - 4 * lev)
            inb = outb
        _, _, tbits, nlt_tot = st

        for q in range(KCH):
            sel_ref[pl.ds(q * 16, 16)] = jnp.broadcast_to(r, (16,)).astype(
                jnp.int32)

        def eb(j, carry):
            clt, ceq = carry
            v = cand_d_ref[pl.ds(j * 16, 16)]
            b = plsc.bitcast(v, jnp.int32)
            iv = cand_i_ref[pl.ds(j * 16, 16)]
            lm = (lane + j * 16) < c
            mlt = (b < tbits) & lm
            plsc.store_compressed(sel_ref.at[pl.ds(clt, 16)], iv, mask=mlt)
            meq = (b == tbits) & lm
            pe = nlt_tot + ceq + plsc.cumsum(meq.astype(jnp.int32)) - 1
            keep = meq & (pe < kk)
            plsc.store_scatter(sel_ref, [pe], iv, mask=keep)
            return (clt + plsc.all_reduce_population_count(mlt)[0],
                    ceq + plsc.all_reduce_population_count(meq)[0])

        lax.fori_loop(0, (c + 15) // 16, eb, (0, 0))

        for q in range(KCH):
            iv = sel_ref[pl.ds(q * 16, 16)]
            for cch in range(4):
                row_ref[pl.ds(cch * K + q * 16, 16)] = plsc.load_gather(
                    ch_ref, [iv + cch * N])
            row_ref[pl.ds(4 * K + q * 16, 16)] = jnp.where(
                iv == r, 1.0, 0.0).astype(jnp.float32)
        pltpu.sync_copy(row_ref, out_hbm.at[r])
        return _carry

    lax.fori_loop(0, ROWS_PER_W, row_body, 0)


def _sc_select(statesT_flat):
    mesh = plsc.VectorSubcoreMesh(core_axis_name="c", subcore_axis_name="s")
    f = pl.kernel(
        _sc_body,
        out_type=jax.ShapeDtypeStruct((N, 5 * K), jnp.float32),
        mesh=mesh,
        compiler_params=pltpu.CompilerParams(needs_layout_passes=False),
        scratch_types=[
            pltpu.VMEM((4 * N,), jnp.float32),
            pltpu.VMEM((N + 16,), jnp.float32),
            pltpu.VMEM((N + 16,), jnp.int32),
            pltpu.VMEM((N + 16,), jnp.float32),
            pltpu.VMEM((N + 16,), jnp.float32),
            pltpu.VMEM((256,), jnp.int32),
            pltpu.VMEM((K + 16,), jnp.int32),
            pltpu.VMEM((5 * K,), jnp.float32),
        ],
    )
    return f(statesT_flat)


def _mlp_body(states_ref, goals_ref, sel_ref, w1_ref, b1_ref, w2_ref, b2_ref,
              wd1a_ref, wd1b_ref, bd1_ref, wd2_ref, bd2_ref, wd3_ref, bd3_ref,
              wd4_ref, bd4_ref, out_ref):
    s = states_ref[...]
    g = goals_ref[...]
    sel = sel_ref[...]
    w1 = w1_ref[...]
    b1 = b1_ref[...]

    x0 = s[:, 0:1] - sel[:, 0, :]
    x1 = s[:, 1:2] - sel[:, 1, :]
    x2 = s[:, 2:3] - sel[:, 2, :]
    x3 = s[:, 3:4] - sel[:, 3, :]
    ind = sel[:, 4, :]
    d2 = x0 * x0 + x1 * x1
    mask = (d2 < OBS_RADIUS_SQ).astype(jnp.float32)

    h1 = (b1.reshape(1, 1, 64)
          + x0[:, :, None] * w1[0, :][None, None, :]
          + x1[:, :, None] * w1[1, :][None, None, :]
          + x2[:, :, None] * w1[2, :][None, None, :]
          + x3[:, :, None] * w1[3, :][None, None, :]
          + ind[:, :, None] * w1[4, :][None, None, :])
    h1 = jnp.maximum(h1, 0.0)
    h1f = h1.reshape(BM * K, 64)

    h2 = jnp.dot(h1f, w2_ref[...], preferred_element_type=jnp.float32)
    h2 = jnp.maximum(h2 + b2_ref[...], 0.0)
    h2 = h2.reshape(BM, K, 128) * mask[:, :, None]
    xl = jnp.max(h2, axis=1)

    e4 = jnp.concatenate([s[:, :2] - g, s[:, 2:]], axis=1)
    d1 = jnp.dot(xl, wd1a_ref[...], preferred_element_type=jnp.float32)
    d1 = d1 + jnp.dot(e4, wd1b_ref[...], preferred_element_type=jnp.float32)
    d1 = jnp.maximum(d1 + bd1_ref[...], 0.0)
    dd = jnp.dot(d1, wd2_ref[...], preferred_element_type=jnp.float32)
    dd = jnp.maximum(dd + bd2_ref[...], 0.0)
    dd = jnp.dot(dd, wd3_ref[...], preferred_element_type=jnp.float32)
    dd = jnp.maximum(dd + bd3_ref[...], 0.0)
    xo = jnp.dot(dd, wd4_ref[...], preferred_element_type=jnp.float32)
    xo = xo + bd4_ref[...]
    sg = 2.0 * jax.nn.sigmoid(xo) + 0.2
    ax = -(sg[:, 0] * e4[:, 0] + sg[:, 1] * e4[:, 2])
    ay = -(sg[:, 2] * e4[:, 1] + sg[:, 3] * e4[:, 3])
    out_ref[...] = jnp.stack([ax, ay], axis=1)


def _full(shape):
    return pl.BlockSpec(shape, lambda i: (0,) * len(shape))


@functools.partial(jax.jit, static_argnames=("interpret",))
def _mlp_call(states, goals, sel5, Wc1, bc1, Wc2, bc2, Wd1, bd1, Wd2, bd2,
              Wd3, bd3, Wd4, bd4, interpret=False):
    grid = (N // BM,)
    return pl.pallas_call(
        _mlp_body,
        grid=grid,
        in_specs=[
            pl.BlockSpec((BM, 4), lambda i: (i, 0)),
            pl.BlockSpec((BM, 2), lambda i: (i, 0)),
            pl.BlockSpec((BM, 5, K), lambda i: (i, 0, 0)),
            _full((5, 64)), _full((1, 64)),
            _full((64, 128)), _full((1, 128)),
            _full((128, 64)), _full((4, 64)), _full((1, 64)),
            _full((64, 128)), _full((1, 128)),
            _full((128, 64)), _full((1, 64)),
            _full((64, 4)), _full((1, 4)),
        ],
        out_specs=pl.BlockSpec((BM, 2), lambda i: (i, 0)),
        out_shape=jax.ShapeDtypeStruct((N, 2), jnp.float32),
        interpret=interpret,
    )(states, goals, sel5, Wc1, bc1.reshape(1, 64), Wc2, bc2.reshape(1, 128),
      Wd1[:128], Wd1[128:], bd1.reshape(1, 64), Wd2, bd2.reshape(1, 128),
      Wd3, bd3.reshape(1, 64), Wd4, bd4.reshape(1, 4))


def kernel(states, goals, Wc1, bc1, Wc2, bc2, Wd1, bd1, Wd2, bd2, Wd3, bd3,
           Wd4, bd4):
    sel5 = _sc_select(states.T.reshape(4 * N)).reshape(N, 5, K)
    return _mlp_call(states, goals, sel5, Wc1, bc1, Wc2, bc2, Wd1, bd1,
                     Wd2, bd2, Wd3, bd3, Wd4, bd4)


# speedup vs baseline: 4.0892x; 1.2188x over previous
import functools

import jax
import jax.numpy as jnp
from jax import lax
from jax.experimental import pallas as pl
from jax.experimental.pallas import tpu as pltpu
from jax.experimental.pallas import tpu_sc as plsc

N = 2048
K = 64
BM = 128
OBS_RADIUS_SQ = 1.0

NW = 32
ROWS_PER_W = N // NW
NCHUNK = N // 16
KCH = K // 16


def _sc_body(statesT_hbm, out_hbm, ch_ref, cand_ref, hist_ref, sel_ref,
             row_ref):
    cid = lax.axis_index("c")
    sid = lax.axis_index("s")
    wid = sid * 2 + cid
    pltpu.sync_copy(statesT_hbm, ch_ref)
    lane = lax.iota(jnp.int32, 16)
    lane16 = lane * 16
    ones = jnp.ones((16,), jnp.int32)
    pad = jnp.full((16,), 0x7FFFFFFF, jnp.int32)

    def row_body(rr, _carry):
        r = wid * ROWS_PER_W + rr
        rsplat = jnp.full((16,), r, jnp.int32)
        xi = plsc.load_gather(ch_ref, [rsplat])
        yi = plsc.load_gather(ch_ref, [rsplat + N])

        def p1(j, cur):
            base = j * 16
            dx = ch_ref[pl.ds(base, 16)] - xi
            dy = ch_ref[pl.ds(N + base, 16)] - yi
            d2 = dx * dx + dy * dy
            m = d2 < OBS_RADIUS_SQ
            ki = (d2 * 2147483648.0).astype(jnp.int32)
            key = (ki & -2048) | (lane + base)
            plsc.store_compressed(cand_ref.at[pl.ds(cur, 16)], key, mask=m)
            return cur + plsc.all_reduce_population_count(m)[0]

        c = lax.fori_loop(0, NCHUNK, p1, 0, unroll=2)
        cand_ref[pl.ds(c, 16)] = pad
        kk = jnp.minimum(K, c)
        for q in range(KCH):
            sel_ref[pl.ds(q * 16, 16)] = rsplat

        def level(state, shift):
            cur_n, want, clt = state

            def run(cur_n, want, clt):
                cc = (cur_n + 15) // 16
                for l in range(16):
                    hist_ref[pl.ds(l * 16, 16)] = jnp.zeros((16,), jnp.int32)

                def hb(j, carry):
                    k = cand_ref[pl.ds(j * 16, 16)]
                    digit = (k >> shift) & 15
                    plsc.addupdate_scatter(hist_ref, [lane16 | digit], ones)
                    return carry

                lax.fori_loop(0, cc, hb, 0)
                tot = hist_ref[pl.ds(0, 16)]
                for l in range(1, 16):
                    tot = tot + hist_ref[pl.ds(l * 16, 16)]
                cum = plsc.cumsum(tot)
                below = cum < want
                bsel = plsc.all_reduce_population_count(below)[0]
                nlt = jnp.sum(jnp.where(below, tot, 0))

                def cb(j, carry):
                    cur2, cl2 = carry
                    k = cand_ref[pl.ds(j * 16, 16)]
                    digit = (k >> shift) & 15
                    meq = digit == bsel
                    mlt = digit < bsel
                    plsc.store_compressed(cand_ref.at[pl.ds(cur2, 16)], k,
                                          mask=meq)
                    plsc.store_compressed(sel_ref.at[pl.ds(cl2, 16)], k,
                                          mask=mlt)
                    return (cur2 + plsc.all_reduce_population_count(meq)[0],
                            cl2 + plsc.all_reduce_population_count(mlt)[0])

                nn, ncl = lax.fori_loop(0, cc, cb, (0, clt))
                cand_ref[pl.ds(nn, 16)] = pad
                return (nn, want - nlt, ncl)

            return lax.cond(cur_n > 1, run, lambda a, b, d: (a, b, d),
                            cur_n, want, clt)

        st = (c, kk, 0)
        for lev in range(8):
            st = level(st, 28 - 4 * lev)
        fin_n, need, clt = st

        k0 = cand_ref[pl.ds(0, 16)]
        mfin = (lane < need) & (lane < fin_n)
        plsc.store_compressed(sel_ref.at[pl.ds(clt, 16)], k0, mask=mfin)

        for q in range(KCH):
            iv = sel_ref[pl.ds(q * 16, 16)] & 2047
            for cch in range(4):
                row_ref[pl.ds(cch * K + q * 16, 16)] = plsc.load_gather(
                    ch_ref, [iv + cch * N])
            row_ref[pl.ds(4 * K + q * 16, 16)] = jnp.where(
                iv == r, 1.0, 0.0).astype(jnp.float32)
        pltpu.sync_copy(row_ref, out_hbm.at[r])
        return _carry

    lax.fori_loop(0, ROWS_PER_W, row_body, 0)


def _sc_select(statesT_flat):
    mesh = plsc.VectorSubcoreMesh(core_axis_name="c", subcore_axis_name="s")
    f = pl.kernel(
        _sc_body,
        out_type=jax.ShapeDtypeStruct((N, 5 * K), jnp.float32),
        mesh=mesh,
        compiler_params=pltpu.CompilerParams(needs_layout_passes=False),
        scratch_types=[
            pltpu.VMEM((4 * N,), jnp.float32),
            pltpu.VMEM((N + 32,), jnp.int32),
            pltpu.VMEM((256,), jnp.int32),
            pltpu.VMEM((K + 16,), jnp.int32),
            pltpu.VMEM((5 * K,), jnp.float32),
        ],
    )
    return f(statesT_flat)


def _mlp_body(states_ref, goals_ref, sel_ref, w1_ref, b1_ref, w2_ref, b2_ref,
              wd1a_ref, wd1b_ref, bd1_ref, wd2_ref, bd2_ref, wd3_ref, bd3_ref,
              wd4_ref, bd4_ref, out_ref):
    s = states_ref[...]
    g = goals_ref[...]
    sel = sel_ref[...]
    w1 = w1_ref[...]
    b1 = b1_ref[...]

    x0 = s[:, 0:1] - sel[:, 0, :]
    x1 = s[:, 1:2] - sel[:, 1, :]
    x2 = s[:, 2:3] - sel[:, 2, :]
    x3 = s[:, 3:4] - sel[:, 3, :]
    ind = sel[:, 4, :]
    d2 = x0 * x0 + x1 * x1
    mask = (d2 < OBS_RADIUS_SQ).astype(jnp.float32)

    h1 = (b1.reshape(1, 1, 64)
          + x0[:, :, None] * w1[0, :][None, None, :]
          + x1[:, :, None] * w1[1, :][None, None, :]
          + x2[:, :, None] * w1[2, :][None, None, :]
          + x3[:, :, None] * w1[3, :][None, None, :]
          + ind[:, :, None] * w1[4, :][None, None, :])
    h1 = jnp.maximum(h1, 0.0)
    h1f = h1.reshape(BM * K, 64)

    h2 = jnp.dot(h1f, w2_ref[...], preferred_element_type=jnp.float32)
    h2 = jnp.maximum(h2 + b2_ref[...], 0.0)
    h2 = h2.reshape(BM, K, 128) * mask[:, :, None]
    xl = jnp.max(h2, axis=1)

    e4 = jnp.concatenate([s[:, :2] - g, s[:, 2:]], axis=1)
    d1 = jnp.dot(xl, wd1a_ref[...], preferred_element_type=jnp.float32)
    d1 = d1 + jnp.dot(e4, wd1b_ref[...], preferred_element_type=jnp.float32)
    d1 = jnp.maximum(d1 + bd1_ref[...], 0.0)
    dd = jnp.dot(d1, wd2_ref[...], preferred_element_type=jnp.float32)
    dd = jnp.maximum(dd + bd2_ref[...], 0.0)
    dd = jnp.dot(dd, wd3_ref[...], preferred_element_type=jnp.float32)
    dd = jnp.maximum(dd + bd3_ref[...], 0.0)
    xo = jnp.dot(dd, wd4_ref[...], preferred_element_type=jnp.float32)
    xo = xo + bd4_ref[...]
    sg = 2.0 * jax.nn.sigmoid(xo) + 0.2
    ax = -(sg[:, 0] * e4[:, 0] + sg[:, 1] * e4[:, 2])
    ay = -(sg[:, 2] * e4[:, 1] + sg[:, 3] * e4[:, 3])
    out_ref[...] = jnp.stack([ax, ay], axis=1)


def _full(shape):
    return pl.BlockSpec(shape, lambda i: (0,) * len(shape))


@functools.partial(jax.jit, static_argnames=("interpret",))
def _mlp_call(states, goals, sel5, Wc1, bc1, Wc2, bc2, Wd1, bd1, Wd2, bd2,
              Wd3, bd3, Wd4, bd4, interpret=False):
    grid = (N // BM,)
    return pl.pallas_call(
        _mlp_body,
        grid=grid,
        in_specs=[
            pl.BlockSpec((BM, 4), lambda i: (i, 0)),
            pl.BlockSpec((BM, 2), lambda i: (i, 0)),
            pl.BlockSpec((BM, 5, K), lambda i: (i, 0, 0)),
            _full((5, 64)), _full((1, 64)),
            _full((64, 128)), _full((1, 128)),
            _full((128, 64)), _full((4, 64)), _full((1, 64)),
            _full((64, 128)), _full((1, 128)),
            _full((128, 64)), _full((1, 64)),
            _full((64, 4)), _full((1, 4)),
        ],
        out_specs=pl.BlockSpec((BM, 2), lambda i: (i, 0)),
        out_shape=jax.ShapeDtypeStruct((N, 2), jnp.float32),
        interpret=interpret,
    )(states, goals, sel5, Wc1, bc1.reshape(1, 64), Wc2, bc2.reshape(1, 128),
      Wd1[:128], Wd1[128:], bd1.reshape(1, 64), Wd2, bd2.reshape(1, 128),
      Wd3, bd3.reshape(1, 64), Wd4, bd4.reshape(1, 4))


def kernel(states, goals, Wc1, bc1, Wc2, bc2, Wd1, bd1, Wd2, bd2, Wd3, bd3,
           Wd4, bd4):
    sel5 = _sc_select(states.T.reshape(4 * N)).reshape(N, 5, K)
    return _mlp_call(states, goals, sel5, Wc1, bc1, Wc2, bc2, Wd1, bd1,
                     Wd2, bd2, Wd3, bd3, Wd4, bd4)


# speedup vs baseline: 4.7458x; 1.1606x over previous
import functools

import jax
import jax.numpy as jnp
from jax import lax
from jax.experimental import pallas as pl
from jax.experimental.pallas import tpu as pltpu
from jax.experimental.pallas import tpu_sc as plsc

N = 2048
K = 64
BM = 128
OBS_RADIUS_SQ = 1.0

NW = 32
ROWS_PER_W = N // NW
NCHUNK = N // 16
KCH = K // 16


def _sc_body(statesT_hbm, out_hbm, ch_ref, cand_ref, hist_ref, sel_ref,
             row_ref):
    cid = lax.axis_index("c")
    sid = lax.axis_index("s")
    wid = sid * 2 + cid
    pltpu.sync_copy(statesT_hbm, ch_ref)
    lane = lax.iota(jnp.int32, 16)
    lane16 = lane * 16
    ones = jnp.ones((16,), jnp.int32)
    pad = jnp.full((16,), 0x7FFFFFFF, jnp.int32)
    for q in range(K * 8 // 16):
        row_ref[pl.ds(q * 16, 16)] = jnp.zeros((16,), jnp.float32)

    def row_body(rr, _carry):
        r = wid * ROWS_PER_W + rr
        rsplat = jnp.full((16,), r, jnp.int32)
        xi = plsc.load_gather(ch_ref, [rsplat])
        yi = plsc.load_gather(ch_ref, [rsplat + N])
        vxi = plsc.load_gather(ch_ref, [rsplat + 2 * N])
        vyi = plsc.load_gather(ch_ref, [rsplat + 3 * N])

        def p1(j, cur):
            base = j * 16
            dx = ch_ref[pl.ds(base, 16)] - xi
            dy = ch_ref[pl.ds(N + base, 16)] - yi
            d2 = dx * dx + dy * dy
            m = d2 < OBS_RADIUS_SQ
            ki = (d2 * 2147483648.0).astype(jnp.int32)
            key = (ki & -2048) | (lane + base)
            plsc.store_compressed(cand_ref.at[pl.ds(cur, 16)], key, mask=m)
            return cur + plsc.all_reduce_population_count(m)[0]

        c = lax.fori_loop(0, NCHUNK, p1, 0, unroll=2)
        cand_ref[pl.ds(c, 16)] = pad
        kk = jnp.minimum(K, c)
        for q in range(KCH):
            sel_ref[pl.ds(q * 16, 16)] = rsplat

        def level(state, shift):
            cur_n, want, clt = state

            def run(cur_n, want, clt):
                cc = (cur_n + 15) // 16
                for l in range(16):
                    hist_ref[pl.ds(l * 16, 16)] = jnp.zeros((16,), jnp.int32)

                def hb(j, carry):
                    k = cand_ref[pl.ds(j * 16, 16)]
                    digit = (k >> shift) & 15
                    plsc.addupdate_scatter(hist_ref, [lane16 | digit], ones)
                    return carry

                lax.fori_loop(0, cc, hb, 0)
                tot = hist_ref[pl.ds(0, 16)]
                for l in range(1, 16):
                    tot = tot + hist_ref[pl.ds(l * 16, 16)]
                cum = plsc.cumsum(tot)
                below = cum < want
                bsel = plsc.all_reduce_population_count(below)[0]
                nlt = jnp.sum(jnp.where(below, tot, 0))

                def cb(j, carry):
                    cur2, cl2 = carry
                    k = cand_ref[pl.ds(j * 16, 16)]
                    digit = (k >> shift) & 15
                    meq = digit == bsel
                    mlt = digit < bsel
                    plsc.store_compressed(cand_ref.at[pl.ds(cur2, 16)], k,
                                          mask=meq)
                    plsc.store_compressed(sel_ref.at[pl.ds(cl2, 16)], k,
                                          mask=mlt)
                    return (cur2 + plsc.all_reduce_population_count(meq)[0],
                            cl2 + plsc.all_reduce_population_count(mlt)[0])

                nn, ncl = lax.fori_loop(0, cc, cb, (0, clt))
                cand_ref[pl.ds(nn, 16)] = pad
                return (nn, want - nlt, ncl)

            return lax.cond(cur_n > 1, run, lambda a, b, d: (a, b, d),
                            cur_n, want, clt)

        st = (c, kk, 0)
        for lev in range(8):
            st = level(st, 28 - 4 * lev)
        fin_n, need, clt = st

        k0 = cand_ref[pl.ds(0, 16)]
        mfin = (lane < need) & (lane < fin_n)
        plsc.store_compressed(sel_ref.at[pl.ds(clt, 16)], k0, mask=mfin)

        one = jnp.ones((16,), jnp.float32)
        zero = jnp.zeros((16,), jnp.float32)
        for q in range(KCH):
            iv = sel_ref[pl.ds(q * 16, 16)] & 2047
            dx = xi - plsc.load_gather(ch_ref, [iv])
            dy = yi - plsc.load_gather(ch_ref, [iv + N])
            dvx = vxi - plsc.load_gather(ch_ref, [iv + 2 * N])
            dvy = vyi - plsc.load_gather(ch_ref, [iv + 3 * N])
            d2s = dx * dx + dy * dy
            mk = jnp.where(d2s < OBS_RADIUS_SQ, one, zero)
            ind = jnp.where(iv == r, one, zero)
            pos = lane * 8 + q * 128
            plsc.store_scatter(row_ref, [pos], dx)
            plsc.store_scatter(row_ref, [pos + 1], dy)
            plsc.store_scatter(row_ref, [pos + 2], dvx)
            plsc.store_scatter(row_ref, [pos + 3], dvy)
            plsc.store_scatter(row_ref, [pos + 4], ind)
            plsc.store_scatter(row_ref, [pos + 5], mk)
        pltpu.sync_copy(row_ref, out_hbm.at[r])
        return _carry

    lax.fori_loop(0, ROWS_PER_W, row_body, 0)


def _sc_select(statesT_flat):
    mesh = plsc.VectorSubcoreMesh(core_axis_name="c", subcore_axis_name="s")
    f = pl.kernel(
        _sc_body,
        out_type=jax.ShapeDtypeStruct((N, K * 8), jnp.float32),
        mesh=mesh,
        compiler_params=pltpu.CompilerParams(needs_layout_passes=False),
        scratch_types=[
            pltpu.VMEM((4 * N,), jnp.float32),
            pltpu.VMEM((N + 32,), jnp.int32),
            pltpu.VMEM((256,), jnp.int32),
            pltpu.VMEM((K + 16,), jnp.int32),
            pltpu.VMEM((K * 8,), jnp.float32),
        ],
    )
    return f(statesT_flat)


def _mlp_body(states_ref, goals_ref, x8_ref, w1_ref, b1_ref, w2_ref, b2_ref,
              wd1a_ref, wd1b_ref, bd1_ref, wd2_ref, bd2_ref, wd3_ref, bd3_ref,
              wd4_ref, bd4_ref, out_ref):
    s = states_ref[...]
    g = goals_ref[...]
    x8 = x8_ref[...]

    h1 = jnp.dot(x8, w1_ref[...], preferred_element_type=jnp.float32)
    h1 = jnp.maximum(h1 + b1_ref[...], 0.0)
    h2 = jnp.dot(h1, w2_ref[...], preferred_element_type=jnp.float32)
    h2 = jnp.maximum(h2 + b2_ref[...], 0.0)
    h2 = h2 * x8[:, 5:6]
    xl = jnp.max(h2.reshape(BM, K, 128), axis=1)

    e4 = jnp.concatenate([s[:, :2] - g, s[:, 2:]], axis=1)
    d1 = jnp.dot(xl, wd1a_ref[...], preferred_element_type=jnp.float32)
    d1 = d1 + jnp.dot(e4, wd1b_ref[...], preferred_element_type=jnp.float32)
    d1 = jnp.maximum(d1 + bd1_ref[...], 0.0)
    dd = jnp.dot(d1, wd2_ref[...], preferred_element_type=jnp.float32)
    dd = jnp.maximum(dd + bd2_ref[...], 0.0)
    dd = jnp.dot(dd, wd3_ref[...], preferred_element_type=jnp.float32)
    dd = jnp.maximum(dd + bd3_ref[...], 0.0)
    xo = jnp.dot(dd, wd4_ref[...], preferred_element_type=jnp.float32)
    xo = xo + bd4_ref[...]
    sg = 2.0 * jax.nn.sigmoid(xo) + 0.2
    ax = -(sg[:, 0] * e4[:, 0] + sg[:, 1] * e4[:, 2])
    ay = -(sg[:, 2] * e4[:, 1] + sg[:, 3] * e4[:, 3])
    out_ref[...] = jnp.stack([ax, ay], axis=1)


def _full(shape):
    return pl.BlockSpec(shape, lambda i: (0,) * len(shape))


@functools.partial(jax.jit, static_argnames=("interpret",))
def _mlp_call(states, goals, x8, Wc1, bc1, Wc2, bc2, Wd1, bd1, Wd2, bd2,
              Wd3, bd3, Wd4, bd4, interpret=False):
    grid = (N // BM,)
    w1pad = jnp.zeros((8, 64), jnp.float32).at[:5].set(Wc1)
    return pl.pallas_call(
        _mlp_body,
        grid=grid,
        in_specs=[
            pl.BlockSpec((BM, 4), lambda i: (i, 0)),
            pl.BlockSpec((BM, 2), lambda i: (i, 0)),
            pl.BlockSpec((BM * K, 8), lambda i: (i, 0)),
            _full((8, 64)), _full((1, 64)),
            _full((64, 128)), _full((1, 128)),
            _full((128, 64)), _full((4, 64)), _full((1, 64)),
            _full((64, 128)), _full((1, 128)),
            _full((128, 64)), _full((1, 64)),
            _full((64, 4)), _full((1, 4)),
        ],
        out_specs=pl.BlockSpec((BM, 2), lambda i: (i, 0)),
        out_shape=jax.ShapeDtypeStruct((N, 2), jnp.float32),
        interpret=interpret,
    )(states, goals, x8, w1pad, bc1.reshape(1, 64), Wc2, bc2.reshape(1, 128),
      Wd1[:128], Wd1[128:], bd1.reshape(1, 64), Wd2, bd2.reshape(1, 128),
      Wd3, bd3.reshape(1, 64), Wd4, bd4.reshape(1, 4))


def kernel(states, goals, Wc1, bc1, Wc2, bc2, Wd1, bd1, Wd2, bd2, Wd3, bd3,
           Wd4, bd4):
    x8 = _sc_select(states.T.reshape(4 * N)).reshape(N * K, 8)
    return _mlp_call(states, goals, x8, Wc1, bc1, Wc2, bc2, Wd1, bd1,
                     Wd2, bd2, Wd3, bd3, Wd4, bd4)


# speedup vs baseline: 5.1967x; 1.0950x over previous
import functools

import jax
import jax.numpy as jnp
from jax import lax
from jax.experimental import pallas as pl
from jax.experimental.pallas import tpu as pltpu
from jax.experimental.pallas import tpu_sc as plsc

N = 2048
K = 64
BM = 128
OBS_RADIUS_SQ = 1.0

NW = 32
NHALF = 2
HROWS = N // NHALF
ROWS_PER_W = HROWS // NW
NCHUNK = N // 16
KCH = K // 16


def _sc_body(half, statesT_hbm, out_hbm, ch_ref, cand_ref, hist_ref, sel_ref,
             row_ref):
    cid = lax.axis_index("c")
    sid = lax.axis_index("s")
    wid = sid * 2 + cid
    pltpu.sync_copy(statesT_hbm, ch_ref)
    lane = lax.iota(jnp.int32, 16)
    lane16 = lane * 16
    ones = jnp.ones((16,), jnp.int32)
    pad = jnp.full((16,), 0x7FFFFFFF, jnp.int32)
    for q in range(K * 8 // 16):
        row_ref[pl.ds(q * 16, 16)] = jnp.zeros((16,), jnp.float32)

    def row_body(rr, _carry):
        r = half * HROWS + wid * ROWS_PER_W + rr
        rsplat = jnp.full((16,), r, jnp.int32)
        xi = plsc.load_gather(ch_ref, [rsplat])
        yi = plsc.load_gather(ch_ref, [rsplat + N])
        vxi = plsc.load_gather(ch_ref, [rsplat + 2 * N])
        vyi = plsc.load_gather(ch_ref, [rsplat + 3 * N])

        def p1(j, cur):
            base = j * 16
            dx = ch_ref[pl.ds(base, 16)] - xi
            dy = ch_ref[pl.ds(N + base, 16)] - yi
            d2 = dx * dx + dy * dy
            m = d2 < OBS_RADIUS_SQ
            ki = (d2 * 2147483648.0).astype(jnp.int32)
            key = (ki & -2048) | (lane + base)
            plsc.store_compressed(cand_ref.at[pl.ds(cur, 16)], key, mask=m)
            return cur + plsc.all_reduce_population_count(m)[0]

        c = lax.fori_loop(0, NCHUNK, p1, 0, unroll=2)
        cand_ref[pl.ds(c, 16)] = pad
        kk = jnp.minimum(K, c)
        for q in range(KCH):
            sel_ref[pl.ds(q * 16, 16)] = rsplat

        def level(state, shift):
            cur_n, want, clt = state

            def run(cur_n, want, clt):
                cc = (cur_n + 15) // 16
                for l in range(16):
                    hist_ref[pl.ds(l * 16, 16)] = jnp.zeros((16,), jnp.int32)

                def hb(j, carry):
                    k = cand_ref[pl.ds(j * 16, 16)]
                    digit = (k >> shift) & 15
                    plsc.addupdate_scatter(hist_ref, [lane16 | digit], ones)
                    return carry

                lax.fori_loop(0, cc, hb, 0)
                tot = hist_ref[pl.ds(0, 16)]
                for l in range(1, 16):
                    tot = tot + hist_ref[pl.ds(l * 16, 16)]
                cum = plsc.cumsum(tot)
                below = cum < want
                bsel = plsc.all_reduce_population_count(below)[0]
                nlt = jnp.sum(jnp.where(below, tot, 0))

                def cb(j, carry):
                    cur2, cl2 = carry
                    k = cand_ref[pl.ds(j * 16, 16)]
                    digit = (k >> shift) & 15
                    meq = digit == bsel
                    mlt = digit < bsel
                    plsc.store_compressed(cand_ref.at[pl.ds(cur2, 16)], k,
                                          mask=meq)
                    plsc.store_compressed(sel_ref.at[pl.ds(cl2, 16)], k,
                                          mask=mlt)
                    return (cur2 + plsc.all_reduce_population_count(meq)[0],
                            cl2 + plsc.all_reduce_population_count(mlt)[0])

                nn, ncl = lax.fori_loop(0, cc, cb, (0, clt))
                cand_ref[pl.ds(nn, 16)] = pad
                return (nn, want - nlt, ncl)

            return lax.cond(cur_n > 1, run, lambda a, b, d: (a, b, d),
                            cur_n, want, clt)

        st = (c, kk, 0)
        for lev in range(8):
            st = level(st, 28 - 4 * lev)
        fin_n, need, clt = st

        k0 = cand_ref[pl.ds(0, 16)]
        mfin = (lane < need) & (lane < fin_n)
        plsc.store_compressed(sel_ref.at[pl.ds(clt, 16)], k0, mask=mfin)

        one = jnp.ones((16,), jnp.float32)
        zero = jnp.zeros((16,), jnp.float32)
        for q in range(KCH):
            iv = sel_ref[pl.ds(q * 16, 16)] & 2047
            dx = xi - plsc.load_gather(ch_ref, [iv])
            dy = yi - plsc.load_gather(ch_ref, [iv + N])
            dvx = vxi - plsc.load_gather(ch_ref, [iv + 2 * N])
            dvy = vyi - plsc.load_gather(ch_ref, [iv + 3 * N])
            d2s = dx * dx + dy * dy
            mk = jnp.where(d2s < OBS_RADIUS_SQ, one, zero)
            ind = jnp.where(iv == r, one, zero)
            pos = lane * 8 + q * 128
            plsc.store_scatter(row_ref, [pos], dx)
            plsc.store_scatter(row_ref, [pos + 1], dy)
            plsc.store_scatter(row_ref, [pos + 2], dvx)
            plsc.store_scatter(row_ref, [pos + 3], dvy)
            plsc.store_scatter(row_ref, [pos + 4], ind)
            plsc.store_scatter(row_ref, [pos + 5], mk)
        pltpu.sync_copy(row_ref, out_hbm.at[r - half * HROWS])
        return _carry

    lax.fori_loop(0, ROWS_PER_W, row_body, 0)


def _sc_select(statesT_flat, half):
    mesh = plsc.VectorSubcoreMesh(core_axis_name="c", subcore_axis_name="s")
    f = pl.kernel(
        functools.partial(_sc_body, half),
        out_type=jax.ShapeDtypeStruct((HROWS, K * 8), jnp.float32),
        mesh=mesh,
        compiler_params=pltpu.CompilerParams(needs_layout_passes=False),
        scratch_types=[
            pltpu.VMEM((4 * N,), jnp.float32),
            pltpu.VMEM((N + 32,), jnp.int32),
            pltpu.VMEM((256,), jnp.int32),
            pltpu.VMEM((K + 16,), jnp.int32),
            pltpu.VMEM((K * 8,), jnp.float32),
        ],
    )
    return f(statesT_flat)


def _mlp_body(states_ref, goals_ref, x8_ref, w1_ref, b1_ref, w2_ref, b2_ref,
              wd1a_ref, wd1b_ref, bd1_ref, wd2_ref, bd2_ref, wd3_ref, bd3_ref,
              wd4_ref, bd4_ref, out_ref):
    s = states_ref[...]
    g = goals_ref[...]
    x8 = x8_ref[...]

    h1 = jnp.dot(x8, w1_ref[...], preferred_element_type=jnp.float32)
    h1 = jnp.maximum(h1 + b1_ref[...], 0.0)
    h2 = jnp.dot(h1, w2_ref[...], preferred_element_type=jnp.float32)
    h2 = jnp.maximum(h2 + b2_ref[...], 0.0)
    h2 = h2 * x8[:, 5:6]
    xl = jnp.max(h2.reshape(BM, K, 128), axis=1)

    e4 = jnp.concatenate([s[:, :2] - g, s[:, 2:]], axis=1)
    d1 = jnp.dot(xl, wd1a_ref[...], preferred_element_type=jnp.float32)
    d1 = d1 + jnp.dot(e4, wd1b_ref[...], preferred_element_type=jnp.float32)
    d1 = jnp.maximum(d1 + bd1_ref[...], 0.0)
    dd = jnp.dot(d1, wd2_ref[...], preferred_element_type=jnp.float32)
    dd = jnp.maximum(dd + bd2_ref[...], 0.0)
    dd = jnp.dot(dd, wd3_ref[...], preferred_element_type=jnp.float32)
    dd = jnp.maximum(dd + bd3_ref[...], 0.0)
    xo = jnp.dot(dd, wd4_ref[...], preferred_element_type=jnp.float32)
    xo = xo + bd4_ref[...]
    sg = 2.0 * jax.nn.sigmoid(xo) + 0.2
    ax = -(sg[:, 0] * e4[:, 0] + sg[:, 1] * e4[:, 2])
    ay = -(sg[:, 2] * e4[:, 1] + sg[:, 3] * e4[:, 3])
    out_ref[...] = jnp.stack([ax, ay], axis=1)


def _full(shape):
    return pl.BlockSpec(shape, lambda i: (0,) * len(shape))


@functools.partial(jax.jit, static_argnames=("interpret",))
def _mlp_call(states, goals, x8, Wc1, bc1, Wc2, bc2, Wd1, bd1, Wd2, bd2,
              Wd3, bd3, Wd4, bd4, interpret=False):
    grid = (HROWS // BM,)
    w1pad = jnp.zeros((8, 64), jnp.float32).at[:5].set(Wc1)
    return pl.pallas_call(
        _mlp_body,
        grid=grid,
        in_specs=[
            pl.BlockSpec((BM, 4), lambda i: (i, 0)),
            pl.BlockSpec((BM, 2), lambda i: (i, 0)),
            pl.BlockSpec((BM * K, 8), lambda i: (i, 0)),
            _full((8, 64)), _full((1, 64)),
            _full((64, 128)), _full((1, 128)),
            _full((128, 64)), _full((4, 64)), _full((1, 64)),
            _full((64, 128)), _full((1, 128)),
            _full((128, 64)), _full((1, 64)),
            _full((64, 4)), _full((1, 4)),
        ],
        out_specs=pl.BlockSpec((BM, 2), lambda i: (i, 0)),
        out_shape=jax.ShapeDtypeStruct((HROWS, 2), jnp.float32),
        interpret=interpret,
    )(states, goals, x8, w1pad, bc1.reshape(1, 64), Wc2, bc2.reshape(1, 128),
      Wd1[:128], Wd1[128:], bd1.reshape(1, 64), Wd2, bd2.reshape(1, 128),
      Wd3, bd3.reshape(1, 64), Wd4, bd4.reshape(1, 4))


def kernel(states, goals, Wc1, bc1, Wc2, bc2, Wd1, bd1, Wd2, bd2, Wd3, bd3,
           Wd4, bd4):
    statesT_flat = states.T.reshape(4 * N)
    outs = []
    for h in range(NHALF):
        x8 = _sc_select(statesT_flat, h).reshape(HROWS * K, 8)
        sl = slice(h * HROWS, (h + 1) * HROWS)
        outs.append(_mlp_call(states[sl], goals[sl], x8, Wc1, bc1, Wc2, bc2,
                              Wd1, bd1, Wd2, bd2, Wd3, bd3, Wd4, bd4))
    return jnp.concatenate(outs, axis=0)
